# baseline (speedup 1.0000x reference)
import jax
import jax.numpy as jnp
from jax import lax
from jax.experimental import pallas as pl
from jax.experimental.pallas import tpu as pltpu
from jax.experimental.pallas import tpu_sc as plsc

_VOCAB = 100000
_EMB = 64
_VPOS = 32
_H = _EMB + _VPOS
_NUM_IMAGE = 5
_B = 1024
_T2 = 52
_CTX = (_NUM_IMAGE - 1) * _T2 + 1
_TGT = _CTX + _T2
_ROWPAD = 264
_LANES = 128
_MASK_ID, _BOS_ID, _EOS_ID = 0, 1, 2
_HB = 512
_HALF = 16
_TCG = 16


def _sc_gather_body(tok_hbm, table_hbm, out_hbm, idx_v, g_bufs, gsem, osem):
    info = plsc.get_sparse_core_info()
    nc = info.num_cores
    nw = nc * info.num_subcores
    bpw = _HB // nw
    wid = lax.axis_index("s") * nc + lax.axis_index("c")
    base = wid * bpw

    pltpu.sync_copy(tok_hbm.at[pl.ds(base, _HALF)], idx_v)

    def gather_batch(i, buf):
        h = lax.rem(i, _HALF)
        pltpu.async_copy(table_hbm.at[idx_v.at[h, 0]],
                         buf.at[pl.ds(0, _LANES)], gsem)
        pltpu.async_copy(table_hbm.at[idx_v.at[h, 1]],
                         buf.at[pl.ds(_LANES, _LANES)], gsem)
        pltpu.async_copy(table_hbm.at[idx_v.at[h, 2, pl.ds(0, 8)]],
                         buf.at[pl.ds(2 * _LANES, 8)], gsem)

    def drain_gather(buf):
        for rows in (_LANES, _LANES, 8):
            pltpu.make_async_copy(
                table_hbm.at[idx_v.at[0, 0, pl.ds(0, rows)]],
                buf.at[pl.ds(0, rows)], gsem).wait()

    def drain_out(buf):
        pltpu.make_async_copy(buf, out_hbm.at[:, 0], osem).wait()

    def with_buf(fn, sel):
        for k in range(2):
            @pl.when(sel == k)
            def _():
                fn(g_bufs[k])

    gather_batch(0, g_bufs[0])

    def per_batch(i, carry):
        cur = lax.rem(i, 2)
        nxt = lax.rem(i + 1, 2)

        with_buf(drain_gather, cur)

        @pl.when(i > 0)
        def _():
            with_buf(drain_out, nxt)

        @pl.when(i + 1 < bpw)
        def _():
            def launch(buf):
                gather_batch(i + 1, buf)
            with_buf(launch, nxt)

        def store(buf):
            pltpu.async_copy(buf, out_hbm.at[:, base + i], osem)
        with_buf(store, cur)
        return carry

    lax.fori_loop(0, bpw, per_batch, 0)
    with_buf(drain_out, lax.rem(bpw - 1, 2))


def _sc_gather(tok_p, table_p):
    mesh = plsc.VectorSubcoreMesh(core_axis_name="c", subcore_axis_name="s")
    fn = pl.kernel(
        _sc_gather_body,
        out_type=jax.ShapeDtypeStruct((_ROWPAD, _HB, _LANES), jnp.float32),
        mesh=mesh,
        scratch_types=[
            pltpu.VMEM((_HALF, 8, _LANES), jnp.int32),
            [pltpu.VMEM((_ROWPAD, _LANES), jnp.float32) for _ in range(2)],
            pltpu.SemaphoreType.DMA,
            pltpu.SemaphoreType.DMA,
        ],
        compiler_params=pltpu.CompilerParams(use_tc_tiling_on_sc=True),
    )
    return fn(tok_p, table_p)


def _tc_add_body(g_ref, bias_ref, out_ref):
    x = jnp.swapaxes(g_ref[...], 1, 2)
    out_ref[...] = x[:, :_H, :] + bias_ref[...][:, :_H, None]


def _tc_add_half0(gath, bias, rows):
    grid = (pl.cdiv(rows, _TCG),)
    return pl.pallas_call(
        _tc_add_body,
        grid=grid,
        in_specs=[
            pl.BlockSpec((_TCG, _HB, _LANES), lambda i: (i, 0, 0)),
            pl.BlockSpec((_TCG, _LANES), lambda i: (i, 0)),
        ],
        out_specs=pl.BlockSpec((_TCG, _H, _HB), lambda i: (i, 0, 0)),
        out_shape=jax.ShapeDtypeStruct((rows, _H, _B), jnp.float32),
        compiler_params=pltpu.CompilerParams(
            dimension_semantics=("arbitrary",)),
    )(gath, bias)


def _tc_add_body_aliased(g_ref, bias_ref, partial_ref, out_ref):
    del partial_ref
    _tc_add_body(g_ref, bias_ref, out_ref)


def _tc_add_half1(gath, bias, partial, rows):
    grid = (pl.cdiv(rows, _TCG),)
    return pl.pallas_call(
        _tc_add_body_aliased,
        grid=grid,
        in_specs=[
            pl.BlockSpec((_TCG, _HB, _LANES), lambda i: (i, 0, 0)),
            pl.BlockSpec((_TCG, _LANES), lambda i: (i, 0)),
            pl.BlockSpec(memory_space=pl.ANY),
        ],
        out_specs=pl.BlockSpec((_TCG, _H, _HB), lambda i: (i, 0, 1)),
        out_shape=jax.ShapeDtypeStruct((rows, _H, _B), jnp.float32),
        input_output_aliases={2: 0},
        compiler_params=pltpu.CompilerParams(
            dimension_semantics=("arbitrary",)),
    )(gath, bias, partial)


def kernel(sequence, table, t_pos_embed, v_pos_embed):
    Bn, Vn, Tn = sequence.shape
    t2 = Tn + 2
    seq = sequence.astype(jnp.int32)
    bos = jnp.full((Bn, Vn, 1), _BOS_ID, jnp.int32)
    eos = jnp.full((Bn, Vn, 1), _EOS_ID, jnp.int32)
    pad_seq = jnp.concatenate([bos, seq, eos], axis=-1)
    tok = jnp.concatenate(
        [pad_seq[:, : Vn - 1].reshape(Bn, (Vn - 1) * t2),
         jnp.full((Bn, 1), _MASK_ID, jnp.int32),
         pad_seq[:, Vn - 1]], axis=1)
    tok_p = jnp.pad(tok, ((0, 0), (0, 8 * _LANES - _TGT))).reshape(Bn, 8, _LANES)

    table_p = jnp.pad(table, ((0, 0), (0, _LANES - _EMB)))

    t52 = t_pos_embed[:t2]
    bias64 = jnp.concatenate(
        [jnp.tile(t52, (Vn - 1, 1)),
         jnp.zeros((1, _EMB), jnp.float32),
         t52,
         jnp.zeros((_ROWPAD - _TGT, _EMB), jnp.float32)], axis=0)
    v5 = v_pos_embed[:Vn]
    vpat = jnp.concatenate(
        [jnp.repeat(v5[: Vn - 1], t2, axis=0),
         jnp.tile(v5[Vn - 1: Vn], (_TGT - (Vn - 1) * t2 + (_ROWPAD - _TGT), 1))],
        axis=0)
    bias = jnp.concatenate(
        [bias64, vpat, jnp.zeros((_ROWPAD, _LANES - _H), jnp.float32)], axis=1)

    gath_a = _sc_gather(tok_p[:_HB], table_p)
    gath_b = _sc_gather(tok_p[_HB:], table_p)
    tgt3a = _tc_add_half0(gath_a, bias, _TGT)
    ctx3a = _tc_add_half0(gath_a, bias, _CTX)
    tgt3 = _tc_add_half1(gath_b, bias, tgt3a, _TGT)
    ctx3 = _tc_add_half1(gath_b, bias, ctx3a, _CTX)
    tgt_emb = jnp.transpose(tgt3, (2, 0, 1))
    ctx_emb = jnp.transpose(ctx3, (2, 0, 1))

    tgt_seq = tok.astype(sequence.dtype)
    ctx_seq = tgt_seq[:, :_CTX]
    return ctx_emb, ctx_seq, tgt_emb, tgt_seq


# speedup vs baseline: 1.0815x; 1.0815x over previous
import jax
import jax.numpy as jnp
from jax import lax
from jax.experimental import pallas as pl
from jax.experimental.pallas import tpu as pltpu
from jax.experimental.pallas import tpu_sc as plsc

_VOCAB = 100000
_EMB = 64
_VPOS = 32
_H = _EMB + _VPOS
_NUM_IMAGE = 5
_B = 1024
_T2 = 52
_CTX = (_NUM_IMAGE - 1) * _T2 + 1
_TGT = _CTX + _T2
_ROWPAD = 264
_LANES = 128
_MASK_ID, _BOS_ID, _EOS_ID = 0, 1, 2
_HALF = 16
_TCG = 16


def _sc_gather_body(tok_hbm, table_hbm, out_hbm, idx_v, g_bufs, gsem, osem):
    info = plsc.get_sparse_core_info()
    nc = info.num_cores
    nw = nc * info.num_subcores
    bpw = _B // nw
    wid = lax.axis_index("s") * nc + lax.axis_index("c")
    base = wid * bpw

    pltpu.sync_copy(tok_hbm.at[pl.ds(base, _HALF)], idx_v)

    def gather_batch(i, buf):
        h = lax.rem(i, _HALF)
        pltpu.async_copy(table_hbm.at[idx_v.at[h, 0]],
                         buf.at[pl.ds(0, _LANES)], gsem)
        pltpu.async_copy(table_hbm.at[idx_v.at[h, 1]],
                         buf.at[pl.ds(_LANES, _LANES)], gsem)
        pltpu.async_copy(table_hbm.at[idx_v.at[h, 2, pl.ds(0, 8)]],
                         buf.at[pl.ds(2 * _LANES, 8)], gsem)

    def drain_gather(buf):
        for rows in (_LANES, _LANES, 8):
            pltpu.make_async_copy(
                table_hbm.at[idx_v.at[0, 0, pl.ds(0, rows)]],
                buf.at[pl.ds(0, rows)], gsem).wait()

    def drain_out(buf):
        pltpu.make_async_copy(buf, out_hbm.at[:, 0], osem).wait()

    def with_buf(fn, sel):
        for k in range(2):
            @pl.when(sel == k)
            def _():
                fn(g_bufs[k])

    gather_batch(0, g_bufs[0])

    def per_batch(i, carry):
        cur = lax.rem(i, 2)
        nxt = lax.rem(i + 1, 2)

        with_buf(drain_gather, cur)

        @pl.when(i + 1 == _HALF)
        def _():
            pltpu.sync_copy(tok_hbm.at[pl.ds(base + _HALF, _HALF)], idx_v)

        @pl.when(i > 0)
        def _():
            with_buf(drain_out, nxt)

        @pl.when(i + 1 < bpw)
        def _():
            def launch(buf):
                gather_batch(i + 1, buf)
            with_buf(launch, nxt)

        def store(buf):
            pltpu.async_copy(buf, out_hbm.at[:, base + i], osem)
        with_buf(store, cur)
        return carry

    lax.fori_loop(0, bpw, per_batch, 0)
    with_buf(drain_out, lax.rem(bpw - 1, 2))


def _sc_gather(tok_p, table_p):
    mesh = plsc.VectorSubcoreMesh(core_axis_name="c", subcore_axis_name="s")
    fn = pl.kernel(
        _sc_gather_body,
        out_type=jax.ShapeDtypeStruct((_ROWPAD, _B, _LANES), jnp.float32),
        mesh=mesh,
        scratch_types=[
            pltpu.VMEM((_HALF, 8, _LANES), jnp.int32),
            [pltpu.VMEM((_ROWPAD, _LANES), jnp.float32) for _ in range(2)],
            pltpu.SemaphoreType.DMA,
            pltpu.SemaphoreType.DMA,
        ],
        compiler_params=pltpu.CompilerParams(use_tc_tiling_on_sc=True),
    )
    return fn(tok_p, table_p)


_CTXBLKS = pl.cdiv(_CTX, _TCG)
_TGTBLKS = pl.cdiv(_TGT, _TCG)


def _tc_add_both_body(g_ref, bias_ref, ctx_ref, tgt_ref):
    x = jnp.swapaxes(g_ref[...], 1, 2)
    e = x[:, :_H, :] + bias_ref[...][:, :_H, None]
    ctx_ref[...] = e
    tgt_ref[...] = e


def _tc_add_tail_body(g_ref, bias_ref, partial_ref, tgt_ref):
    del partial_ref
    x = jnp.swapaxes(g_ref[...], 1, 2)
    tgt_ref[...] = x[:, :_H, :] + bias_ref[...][:, :_H, None]


def _tc_add(gath, bias):
    ctx3, tgt_head = pl.pallas_call(
        _tc_add_both_body,
        grid=(_CTXBLKS,),
        in_specs=[
            pl.BlockSpec((_TCG, _B, _LANES), lambda i: (i, 0, 0)),
            pl.BlockSpec((_TCG, _LANES), lambda i: (i, 0)),
        ],
        out_specs=[
            pl.BlockSpec((_TCG, _H, _B), lambda i: (i, 0, 0)),
            pl.BlockSpec((_TCG, _H, _B), lambda i: (i, 0, 0)),
        ],
        out_shape=[
            jax.ShapeDtypeStruct((_CTX, _H, _B), jnp.float32),
            jax.ShapeDtypeStruct((_TGT, _H, _B), jnp.float32),
        ],
        compiler_params=pltpu.CompilerParams(
            dimension_semantics=("arbitrary",)),
    )(gath, bias)
    tgt3 = pl.pallas_call(
        _tc_add_tail_body,
        grid=(_TGTBLKS - _CTXBLKS,),
        in_specs=[
            pl.BlockSpec((_TCG, _B, _LANES), lambda i: (i + _CTXBLKS, 0, 0)),
            pl.BlockSpec((_TCG, _LANES), lambda i: (i + _CTXBLKS, 0)),
            pl.BlockSpec(memory_space=pl.ANY),
        ],
        out_specs=pl.BlockSpec((_TCG, _H, _B), lambda i: (i + _CTXBLKS, 0, 0)),
        out_shape=jax.ShapeDtypeStruct((_TGT, _H, _B), jnp.float32),
        input_output_aliases={2: 0},
        compiler_params=pltpu.CompilerParams(
            dimension_semantics=("arbitrary",)),
    )(gath, bias, tgt_head)
    return ctx3, tgt3


def kernel(sequence, table, t_pos_embed, v_pos_embed):
    Bn, Vn, Tn = sequence.shape
    t2 = Tn + 2
    seq = sequence.astype(jnp.int32)
    bos = jnp.full((Bn, Vn, 1), _BOS_ID, jnp.int32)
    eos = jnp.full((Bn, Vn, 1), _EOS_ID, jnp.int32)
    pad_seq = jnp.concatenate([bos, seq, eos], axis=-1)
    tok = jnp.concatenate(
        [pad_seq[:, : Vn - 1].reshape(Bn, (Vn - 1) * t2),
         jnp.full((Bn, 1), _MASK_ID, jnp.int32),
         pad_seq[:, Vn - 1]], axis=1)
    tok_p = jnp.pad(tok, ((0, 0), (0, 8 * _LANES - _TGT))).reshape(Bn, 8, _LANES)

    table_p = jnp.pad(table, ((0, 0), (0, _LANES - _EMB)))

    t52 = t_pos_embed[:t2]
    bias64 = jnp.concatenate(
        [jnp.tile(t52, (Vn - 1, 1)),
         jnp.zeros((1, _EMB), jnp.float32),
         t52,
         jnp.zeros((_ROWPAD - _TGT, _EMB), jnp.float32)], axis=0)
    v5 = v_pos_embed[:Vn]
    vpat = jnp.concatenate(
        [jnp.repeat(v5[: Vn - 1], t2, axis=0),
         jnp.tile(v5[Vn - 1: Vn], (_TGT - (Vn - 1) * t2 + (_ROWPAD - _TGT), 1))],
        axis=0)
    bias = jnp.concatenate(
        [bias64, vpat, jnp.zeros((_ROWPAD, _LANES - _H), jnp.float32)], axis=1)

    gath = _sc_gather(tok_p, table_p)
    ctx3, tgt3 = _tc_add(gath, bias)
    tgt_emb = jnp.transpose(tgt3, (2, 0, 1))
    ctx_emb = jnp.transpose(ctx3, (2, 0, 1))

    tgt_seq = tok.astype(sequence.dtype)
    ctx_seq = tgt_seq[:, :_CTX]
    return ctx_emb, ctx_seq, tgt_emb, tgt_seq
